# baseline (speedup 1.0000x reference)
import jax
import jax.numpy as jnp
from jax import lax
from jax.experimental import pallas as pl
from jax.experimental.pallas import tpu as pltpu
from jax.experimental.pallas import tpu_sc as plsc

_VOCAB = 100000
_PAD = 1
_LANES = 16


def _sc_body(ids_hbm, beta_hbm, out_hbm, ids_v, row_v, vals_v, beta_v):
    nc = 2
    rows_per_w, lp = ids_v.shape
    groups = lp // _LANES
    wid = lax.axis_index("s") * nc + lax.axis_index("c")
    base = wid * rows_per_w

    pltpu.sync_copy(ids_hbm.at[pl.ds(base, rows_per_w)], ids_v)
    pltpu.sync_copy(beta_hbm, beta_v)
    escale = jnp.exp(beta_v[...])

    zeros16 = jnp.zeros((_LANES,), jnp.float32)
    ones16 = jnp.ones((_LANES,), jnp.float32)

    def _zero(i, _):
        row_v[pl.ds(i * _LANES, _LANES)] = zeros16
        return _

    lax.fori_loop(0, _VOCAB // _LANES, _zero, None)

    def _row(r, _):
        def _add(j, _c):
            idx = ids_v[r, pl.ds(j * _LANES, _LANES)]
            plsc.addupdate_scatter(row_v, [idx], ones16)
            return _c

        lax.fori_loop(0, groups, _add, None)

        def _gather(j, _c):
            idx = ids_v[r, pl.ds(j * _LANES, _LANES)]
            c = plsc.load_gather(row_v, [idx])
            v = c / (c + escale)
            v = jnp.where(idx == _PAD, 0.0, v)
            vals_v[pl.ds(j * _LANES, _LANES)] = v
            return _c

        lax.fori_loop(0, groups, _gather, None)

        def _scat(j, _c):
            idx = ids_v[r, pl.ds(j * _LANES, _LANES)]
            plsc.store_scatter(row_v, [idx], vals_v[pl.ds(j * _LANES, _LANES)])
            return _c

        lax.fori_loop(0, groups, _scat, None)

        pltpu.sync_copy(row_v, out_hbm.at[base + r])

        def _restore(j, _c):
            idx = ids_v[r, pl.ds(j * _LANES, _LANES)]
            plsc.store_scatter(row_v, [idx], zeros16)
            return _c

        lax.fori_loop(0, groups, _restore, None)
        return _

    lax.fori_loop(0, rows_per_w, _row, None)


def kernel(input_ids, beta):
    b, l = input_ids.shape
    nw = 32
    nsplit = 4
    rows_per_w = b // (nw * nsplit)
    lp = -(-l // _LANES) * _LANES
    ids = jnp.pad(input_ids, ((0, 0), (0, lp - l)), constant_values=_PAD)
    beta_vec = jnp.broadcast_to(beta.astype(jnp.float32), (_LANES,))

    mesh = plsc.VectorSubcoreMesh(core_axis_name="c", subcore_axis_name="s")
    run = pl.kernel(
        _sc_body,
        out_type=jax.ShapeDtypeStruct((b // nsplit, _VOCAB), jnp.float32),
        mesh=mesh,
        scratch_types=[
            pltpu.VMEM((rows_per_w, lp), jnp.int32),
            pltpu.VMEM((_VOCAB,), jnp.float32),
            pltpu.VMEM((lp,), jnp.float32),
            pltpu.VMEM((_LANES,), jnp.float32),
        ],
        compiler_params=pltpu.CompilerParams(needs_layout_passes=False),
    )
    rows = b // nsplit
    parts = [
        run(lax.slice_in_dim(ids, s * rows, (s + 1) * rows, axis=0), beta_vec)
        for s in range(nsplit)
    ]
    return jnp.concatenate(parts, axis=0)


# speedup vs baseline: 1.5668x; 1.5668x over previous
import jax
import jax.numpy as jnp
from jax import lax
from jax.experimental import pallas as pl
from jax.experimental.pallas import tpu as pltpu
from jax.experimental.pallas import tpu_sc as plsc

_VOCAB = 100000
_PAD = 1
_LANES = 16


def _sc_body(ids_hbm, beta_hbm, out_hbm, ids_v, row_v, vals_v, beta_v):
    nc = 2
    rows_per_w, lp = ids_v.shape
    groups = lp // _LANES
    wid = lax.axis_index("s") * nc + lax.axis_index("c")
    base = wid * rows_per_w

    pltpu.sync_copy(ids_hbm.at[pl.ds(base, rows_per_w)], ids_v)
    pltpu.sync_copy(beta_hbm, beta_v)
    escale = jnp.exp(beta_v[...])

    zeros16 = jnp.zeros((_LANES,), jnp.float32)
    ones16 = jnp.ones((_LANES,), jnp.float32)

    def _zero(i, _):
        row_v[pl.ds(i * _LANES, _LANES)] = zeros16
        return _

    lax.fori_loop(0, _VOCAB // _LANES, _zero, None)

    def _row(r, _):
        def _add(j, _c):
            idx = ids_v[r, pl.ds(j * _LANES, _LANES)]
            plsc.addupdate_scatter(row_v, [idx], ones16)
            return _c

        lax.fori_loop(0, groups, _add, None)

        def _gather(j, _c):
            idx = ids_v[r, pl.ds(j * _LANES, _LANES)]
            c = plsc.load_gather(row_v, [idx])
            v = c / (c + escale)
            v = jnp.where(idx == _PAD, 0.0, v)
            vals_v[pl.ds(j * _LANES, _LANES)] = v
            return _c

        lax.fori_loop(0, groups, _gather, None)

        def _scat(j, _c):
            idx = ids_v[r, pl.ds(j * _LANES, _LANES)]
            plsc.store_scatter(row_v, [idx], vals_v[pl.ds(j * _LANES, _LANES)])
            return _c

        lax.fori_loop(0, groups, _scat, None)

        pltpu.sync_copy(row_v, out_hbm.at[base + r])

        def _restore(j, _c):
            idx = ids_v[r, pl.ds(j * _LANES, _LANES)]
            plsc.store_scatter(row_v, [idx], zeros16)
            return _c

        lax.fori_loop(0, groups, _restore, None)
        return _

    lax.fori_loop(0, rows_per_w, _row, None)


def kernel(input_ids, beta):
    b, l = input_ids.shape
    nw = 32
    nsplit = 1
    rows_per_w = b // (nw * nsplit)
    lp = -(-l // _LANES) * _LANES
    ids = jnp.pad(input_ids, ((0, 0), (0, lp - l)), constant_values=_PAD)
    beta_vec = jnp.broadcast_to(beta.astype(jnp.float32), (_LANES,))

    mesh = plsc.VectorSubcoreMesh(core_axis_name="c", subcore_axis_name="s")
    run = pl.kernel(
        _sc_body,
        out_type=jax.ShapeDtypeStruct((b // nsplit, _VOCAB), jnp.float32),
        mesh=mesh,
        scratch_types=[
            pltpu.VMEM((rows_per_w, lp), jnp.int32),
            pltpu.VMEM((_VOCAB,), jnp.float32),
            pltpu.VMEM((lp,), jnp.float32),
            pltpu.VMEM((_LANES,), jnp.float32),
        ],
        compiler_params=pltpu.CompilerParams(needs_layout_passes=False),
    )
    rows = b // nsplit
    parts = [
        run(lax.slice_in_dim(ids, s * rows, (s + 1) * rows, axis=0), beta_vec)
        for s in range(nsplit)
    ]
    return jnp.concatenate(parts, axis=0)
